# baseline (speedup 1.0000x reference)
import functools
import jax
import jax.numpy as jnp
from jax import lax
from jax.experimental import pallas as pl
from jax.experimental.pallas import tpu as pltpu
from jax.experimental.pallas import tpu_sc as plsc

NC = 2
NS = 16
NW = NC * NS
CH = 32


def _sc_row_sigmoid(rows_total, base_row, K):
    rows_w = rows_total // NW
    n_iter = rows_w // CH
    NG = CH // 16
    NJ = K // 16
    mesh = plsc.VectorSubcoreMesh(core_axis_name="c", subcore_axis_name="s")

    @functools.partial(
        pl.kernel,
        out_type=jax.ShapeDtypeStruct((rows_total,), jnp.float32),
        mesh=mesh,
        scratch_types=[
            pltpu.VMEM((2, CH, K), jnp.float32),
            pltpu.VMEM((K,), jnp.float32),
            pltpu.VMEM((16,), jnp.float32),
            pltpu.VMEM((rows_w,), jnp.float32),
            pltpu.VMEM((256,), jnp.float32),
            pltpu.SemaphoreType.DMA((2,)),
        ],
        compiler_params=pltpu.CompilerParams(needs_layout_passes=False),
    )
    def sc_fn(x_hbm, w_hbm, b_hbm, o_hbm, xbuf, wbuf, bbuf, obuf, tbuf, xsems):
        wid = lax.axis_index("s") * NC + lax.axis_index("c")
        row0 = base_row + wid * rows_w

        pltpu.sync_copy(w_hbm, wbuf)
        pltpu.sync_copy(b_hbm, bbuf)
        bvec = bbuf[...]

        def start(it, slot):
            pltpu.make_async_copy(
                x_hbm.at[pl.ds(row0 + it * CH, CH), :],
                xbuf.at[slot],
                xsems.at[slot],
            ).start()

        def wait(it, slot):
            pltpu.make_async_copy(
                x_hbm.at[pl.ds(row0 + it * CH, CH), :],
                xbuf.at[slot],
                xsems.at[slot],
            ).wait()

        start(0, 0)
        start(1, 1)

        iota = lax.iota(jnp.int32, 16)

        def super_body(s, carry):
            for b in range(2):
                it = 2 * s + b
                wait(it, b)
                for g in range(NG):
                    accs = tuple(
                        jnp.zeros((16,), jnp.float32) for _ in range(16))

                    def jbody(j, accs, b=b, g=g):
                        wj = wbuf[pl.ds(j * 16, 16)]
                        return tuple(
                            accs[r] + xbuf[b, g * 16 + r, pl.ds(j * 16, 16)] * wj
                            for r in range(16))

                    accs = lax.fori_loop(0, NJ, jbody, accs)
                    for r in range(16):
                        tbuf[pl.ds(r * 16, 16)] = accs[r]
                    scaled = iota * 16
                    acc = jnp.zeros((16,), jnp.float32)
                    for k in range(16):
                        acc = acc + plsc.load_gather(tbuf, [scaled + k])
                    z = acc - bvec
                    y = 1.0 / (1.0 + jnp.exp(-z))
                    obuf[pl.ds(it * CH + g * 16, 16)] = y

                @pl.when(it + 2 < n_iter)
                def _(it=it, b=b):
                    start(it + 2, b)
            return carry

        lax.fori_loop(0, n_iter // 2, super_body, 0)
        pltpu.sync_copy(obuf, o_hbm.at[pl.ds(row0, rows_w)])

    return sc_fn


def kernel(x, input_weights, bias):
    B, K = x.shape
    w1d = input_weights.reshape(K)
    b16 = jnp.broadcast_to(bias, (16,))
    y = _sc_row_sigmoid(B, 0, K)(x, w1d, b16)
    return y.reshape(B, 1)


# speedup vs baseline: 1.1350x; 1.1350x over previous
import functools
import jax
import jax.numpy as jnp
from jax import lax
from jax.experimental import pallas as pl
from jax.experimental.pallas import tpu as pltpu
from jax.experimental.pallas import tpu_sc as plsc

NC = 2
NS = 16
NW = NC * NS
CH = 32
BM = 2048
TC_ROWS = 36864


def _sc_row_sigmoid(rows_total, base_row, K):
    rows_w = rows_total // NW
    n_iter = rows_w // CH
    NG = CH // 16
    NJ = K // 16
    mesh = plsc.VectorSubcoreMesh(core_axis_name="c", subcore_axis_name="s")

    @functools.partial(
        pl.kernel,
        out_type=jax.ShapeDtypeStruct((rows_total,), jnp.float32),
        mesh=mesh,
        scratch_types=[
            pltpu.VMEM((2, CH, K), jnp.float32),
            pltpu.VMEM((K,), jnp.float32),
            pltpu.VMEM((16,), jnp.float32),
            pltpu.VMEM((rows_w,), jnp.float32),
            pltpu.VMEM((256,), jnp.float32),
            pltpu.SemaphoreType.DMA((2,)),
        ],
        compiler_params=pltpu.CompilerParams(needs_layout_passes=False),
    )
    def sc_fn(x_hbm, w_hbm, b_hbm, o_hbm, xbuf, wbuf, bbuf, obuf, tbuf, xsems):
        wid = lax.axis_index("s") * NC + lax.axis_index("c")
        row0 = base_row + wid * rows_w

        pltpu.sync_copy(w_hbm, wbuf)
        pltpu.sync_copy(b_hbm, bbuf)
        bvec = bbuf[...]

        def start(it, slot):
            pltpu.make_async_copy(
                x_hbm.at[pl.ds(row0 + it * CH, CH), :],
                xbuf.at[slot],
                xsems.at[slot],
            ).start()

        def wait(it, slot):
            pltpu.make_async_copy(
                x_hbm.at[pl.ds(row0 + it * CH, CH), :],
                xbuf.at[slot],
                xsems.at[slot],
            ).wait()

        start(0, 0)
        start(1, 1)

        iota = lax.iota(jnp.int32, 16)

        def super_body(s, carry):
            for b in range(2):
                it = 2 * s + b
                wait(it, b)
                for g in range(NG):
                    accs = tuple(
                        jnp.zeros((16,), jnp.float32) for _ in range(16))

                    def jbody(j, accs, b=b, g=g):
                        wj = wbuf[pl.ds(j * 16, 16)]
                        return tuple(
                            accs[r] + xbuf[b, g * 16 + r, pl.ds(j * 16, 16)] * wj
                            for r in range(16))

                    accs = lax.fori_loop(0, NJ, jbody, accs)
                    for r in range(16):
                        tbuf[pl.ds(r * 16, 16)] = accs[r]
                    scaled = iota * 16
                    acc = jnp.zeros((16,), jnp.float32)
                    for k in range(16):
                        acc = acc + plsc.load_gather(tbuf, [scaled + k])
                    z = acc - bvec
                    y = 1.0 / (1.0 + jnp.exp(-z))
                    obuf[pl.ds(it * CH + g * 16, 16)] = y

                @pl.when(it + 2 < n_iter)
                def _(it=it, b=b):
                    start(it + 2, b)
            return carry

        lax.fori_loop(0, n_iter // 2, super_body, 0)
        pltpu.sync_copy(obuf, o_hbm.at[pl.ds(row0, rows_w)])

    return sc_fn


def _tc_body(x_ref, w_ref, b_ref, o_ref):
    wx = jax.lax.dot_general(
        x_ref[...], w_ref[...], (((1,), (1,)), ((), ())),
        preferred_element_type=jnp.float32)
    o_ref[...] = jax.nn.sigmoid(wx - b_ref[0])


def _tc_row_sigmoid(x, input_weights, bias, rows):
    B, K = x.shape
    return pl.pallas_call(
        _tc_body,
        grid=(rows // BM,),
        in_specs=[
            pl.BlockSpec((BM, K), lambda i: (i, 0)),
            pl.BlockSpec((1, K), lambda i: (0, 0)),
            pl.BlockSpec(memory_space=pltpu.SMEM),
        ],
        out_specs=pl.BlockSpec((BM, 1), lambda i: (i, 0)),
        out_shape=jax.ShapeDtypeStruct((rows, 1), jnp.float32),
    )(x, input_weights, bias)


def kernel(x, input_weights, bias):
    B, K = x.shape
    w1d = input_weights.reshape(K)
    b16 = jnp.broadcast_to(bias, (16,))
    y_sc = _sc_row_sigmoid(B - TC_ROWS, TC_ROWS, K)(x, w1d, b16)
    y_tc = _tc_row_sigmoid(x, input_weights, bias, TC_ROWS)
    return jnp.concatenate([y_tc, y_sc.reshape(B - TC_ROWS, 1)], axis=0)


# speedup vs baseline: 1.1373x; 1.0020x over previous
import functools
import jax
import jax.numpy as jnp
from jax import lax
from jax.experimental import pallas as pl
from jax.experimental.pallas import tpu as pltpu
from jax.experimental.pallas import tpu_sc as plsc

NC = 2
NS = 16
NW = NC * NS
CH = 32
BM = 2048
TC_ROWS = 36864


def _sc_row_sigmoid(rows_total, base_row, K):
    rows_w = rows_total // NW
    n_iter = rows_w // CH
    NG = CH // 16
    NJ = K // 16
    mesh = plsc.VectorSubcoreMesh(core_axis_name="c", subcore_axis_name="s")

    @functools.partial(
        pl.kernel,
        out_type=jax.ShapeDtypeStruct((rows_total,), jnp.float32),
        mesh=mesh,
        scratch_types=[
            pltpu.VMEM((2, CH, K), jnp.float32),
            pltpu.VMEM((K,), jnp.float32),
            pltpu.VMEM((16,), jnp.float32),
            pltpu.VMEM((rows_w,), jnp.float32),
            pltpu.VMEM((256,), jnp.float32),
            pltpu.SemaphoreType.DMA((2,)),
        ],
        compiler_params=pltpu.CompilerParams(needs_layout_passes=False),
    )
    def sc_fn(x_hbm, w_hbm, b_hbm, o_hbm, xbuf, wbuf, bbuf, obuf, tbuf, xsems):
        wid = lax.axis_index("s") * NC + lax.axis_index("c")
        row0 = base_row + wid * rows_w

        pltpu.sync_copy(w_hbm, wbuf)
        pltpu.sync_copy(b_hbm, bbuf)
        bvec = bbuf[...]

        def start(it, slot):
            pltpu.make_async_copy(
                x_hbm.at[pl.ds(row0 + it * CH, CH), :],
                xbuf.at[slot],
                xsems.at[slot],
            ).start()

        def wait(it, slot):
            pltpu.make_async_copy(
                x_hbm.at[pl.ds(row0 + it * CH, CH), :],
                xbuf.at[slot],
                xsems.at[slot],
            ).wait()

        start(0, 0)
        start(1, 1)

        iota = lax.iota(jnp.int32, 16)

        def super_body(s, carry):
            for b in range(2):
                it = 2 * s + b
                wait(it, b)
                for g in range(NG):
                    accs = tuple(
                        jnp.zeros((16,), jnp.float32) for _ in range(16))

                    def jbody(j, accs, b=b, g=g):
                        wj = wbuf[pl.ds(j * 16, 16)]
                        return tuple(
                            accs[r] + xbuf[b, g * 16 + r, pl.ds(j * 16, 16)] * wj
                            for r in range(16))

                    accs = lax.fori_loop(0, NJ, jbody, accs)
                    for r in range(16):
                        tbuf[pl.ds(r * 16, 16)] = accs[r]
                    scaled = iota * 16
                    acc = jnp.zeros((16,), jnp.float32)
                    for k in range(16):
                        acc = acc + plsc.load_gather(tbuf, [scaled + k])
                    z = acc - bvec
                    y = 1.0 / (1.0 + jnp.exp(-z))
                    obuf[pl.ds(it * CH + g * 16, 16)] = y

                @pl.when(it + 2 < n_iter)
                def _(it=it, b=b):
                    start(it + 2, b)
            return carry

        lax.fori_loop(0, n_iter // 2, super_body, 0)
        pltpu.sync_copy(obuf, o_hbm.at[pl.ds(row0, rows_w)])

    return sc_fn


def _tc_body(x_ref, w_ref, b_ref, o_ref):
    wx = jax.lax.dot_general(
        x_ref[...], w_ref[...], (((1,), (1,)), ((), ())),
        preferred_element_type=jnp.float32)
    o_ref[...] = jax.nn.sigmoid(wx - b_ref[0])


def _tc_row_sigmoid(x, input_weights, bias, rows, row_base):
    B, K = x.shape
    off = row_base // BM
    return pl.pallas_call(
        _tc_body,
        grid=(rows // BM,),
        in_specs=[
            pl.BlockSpec((BM, K), lambda i: (i + off, 0)),
            pl.BlockSpec((1, K), lambda i: (0, 0)),
            pl.BlockSpec(memory_space=pltpu.SMEM),
        ],
        out_specs=pl.BlockSpec((BM, 1), lambda i: (i, 0)),
        out_shape=jax.ShapeDtypeStruct((rows, 1), jnp.float32),
    )(x, input_weights, bias)


def kernel(x, input_weights, bias):
    B, K = x.shape
    sc_rows = B - TC_ROWS
    w1d = input_weights.reshape(K)
    b16 = jnp.broadcast_to(bias, (16,))
    y_sc = _sc_row_sigmoid(sc_rows, 0, K)(x, w1d, b16)
    y_tc = _tc_row_sigmoid(x, input_weights, bias, TC_ROWS, sc_rows)
    return jnp.concatenate([y_sc.reshape(sc_rows, 1), y_tc], axis=0)


# speedup vs baseline: 1.6515x; 1.4521x over previous
import functools
import jax
import jax.numpy as jnp
from jax import lax
from jax.experimental import pallas as pl
from jax.experimental.pallas import tpu as pltpu
from jax.experimental.pallas import tpu_sc as plsc

NC = 2
NS = 16
NW = NC * NS
CH = 32
BM = 2048
TC_ROWS = 65536


def _sc_row_sigmoid(rows_total, base_row, K):
    rows_w = rows_total // NW
    n_iter = rows_w // CH
    NG = CH // 16
    NJ = K // 16
    mesh = plsc.VectorSubcoreMesh(core_axis_name="c", subcore_axis_name="s")

    @functools.partial(
        pl.kernel,
        out_type=jax.ShapeDtypeStruct((rows_total,), jnp.float32),
        mesh=mesh,
        scratch_types=[
            pltpu.VMEM((2, CH, K), jnp.float32),
            pltpu.VMEM((K,), jnp.float32),
            pltpu.VMEM((16,), jnp.float32),
            pltpu.VMEM((rows_w,), jnp.float32),
            pltpu.VMEM((256,), jnp.float32),
            pltpu.SemaphoreType.DMA((2,)),
        ],
        compiler_params=pltpu.CompilerParams(needs_layout_passes=False),
    )
    def sc_fn(x_hbm, w_hbm, b_hbm, o_hbm, xbuf, wbuf, bbuf, obuf, tbuf, xsems):
        wid = lax.axis_index("s") * NC + lax.axis_index("c")
        row0 = base_row + wid * rows_w

        pltpu.sync_copy(w_hbm, wbuf)
        pltpu.sync_copy(b_hbm, bbuf)
        bvec = bbuf[...]

        def start(it, slot):
            pltpu.make_async_copy(
                x_hbm.at[pl.ds(row0 + it * CH, CH), :],
                xbuf.at[slot],
                xsems.at[slot],
            ).start()

        def wait(it, slot):
            pltpu.make_async_copy(
                x_hbm.at[pl.ds(row0 + it * CH, CH), :],
                xbuf.at[slot],
                xsems.at[slot],
            ).wait()

        start(0, 0)
        start(1, 1)

        iota = lax.iota(jnp.int32, 16)

        def super_body(s, carry):
            for b in range(2):
                it = 2 * s + b
                wait(it, b)
                for g in range(NG):
                    accs = tuple(
                        jnp.zeros((16,), jnp.float32) for _ in range(16))

                    def jbody(j, accs, b=b, g=g):
                        wj = wbuf[pl.ds(j * 16, 16)]
                        return tuple(
                            accs[r] + xbuf[b, g * 16 + r, pl.ds(j * 16, 16)] * wj
                            for r in range(16))

                    accs = lax.fori_loop(0, NJ, jbody, accs)
                    for r in range(16):
                        tbuf[pl.ds(r * 16, 16)] = accs[r]
                    scaled = iota * 16
                    acc = jnp.zeros((16,), jnp.float32)
                    for k in range(16):
                        acc = acc + plsc.load_gather(tbuf, [scaled + k])
                    z = acc - bvec
                    y = 1.0 / (1.0 + jnp.exp(-z))
                    obuf[pl.ds(it * CH + g * 16, 16)] = y

                @pl.when(it + 2 < n_iter)
                def _(it=it, b=b):
                    start(it + 2, b)
            return carry

        lax.fori_loop(0, n_iter // 2, super_body, 0)
        pltpu.sync_copy(obuf, o_hbm.at[pl.ds(row0, rows_w)])

    return sc_fn


def _tc_body(x_ref, w_ref, b_ref, o_ref):
    wx = jax.lax.dot_general(
        w_ref[...], x_ref[...], (((1,), (1,)), ((), ())),
        preferred_element_type=jnp.float32)
    o_ref[0] = jax.nn.sigmoid(wx - b_ref[0])


def _tc_row_sigmoid(x, input_weights, bias, rows, row_base):
    B, K = x.shape
    off = row_base // BM
    out = pl.pallas_call(
        _tc_body,
        grid=(rows // BM,),
        in_specs=[
            pl.BlockSpec((BM, K), lambda i: (i + off, 0)),
            pl.BlockSpec((1, K), lambda i: (0, 0)),
            pl.BlockSpec(memory_space=pltpu.SMEM),
        ],
        out_specs=pl.BlockSpec((1, 1, BM), lambda i: (i, 0, 0)),
        out_shape=jax.ShapeDtypeStruct((rows // BM, 1, BM), jnp.float32),
    )(x, input_weights, bias)
    return out.reshape(rows, 1)


def kernel(x, input_weights, bias):
    B, K = x.shape
    sc_rows = B - TC_ROWS
    parts = []
    if sc_rows:
        w1d = input_weights.reshape(K)
        b16 = jnp.broadcast_to(bias, (16,))
        y_sc = _sc_row_sigmoid(sc_rows, 0, K)(x, w1d, b16)
        parts.append(y_sc.reshape(sc_rows, 1))
    if TC_ROWS:
        parts.append(_tc_row_sigmoid(x, input_weights, bias, TC_ROWS, sc_rows))
    if len(parts) == 1:
        return parts[0]
    return jnp.concatenate(parts, axis=0)
